# Initial kernel scaffold; baseline (speedup 1.0000x reference)
import jax
import jax.numpy as jnp
from jax.experimental import pallas as pl


def kernel(token_ids, W, b):
    raise NotImplementedError("write your pallas kernel here")


# speedup vs baseline: 5.3802x; 5.3802x over previous
import functools

import jax
import jax.numpy as jnp
from jax import lax
from jax.experimental import pallas as pl
from jax.experimental.pallas import tpu as pltpu
from jax.experimental.pallas import tpu_sc as plsc

B, L, V, D = 1024, 200, 100000, 64
LANE = 16
G = D // LANE
NC, NS = 2, 16
NW = NC * NS
RPW = B // NW
LCH = 100
NCH = L // LCH


def _bag_kernel(tok3, wt, bias):
    mesh = plsc.VectorSubcoreMesh(core_axis_name="c", subcore_axis_name="s")

    @functools.partial(
        pl.kernel,
        out_type=jax.ShapeDtypeStruct((B, D), jnp.float32),
        mesh=mesh,
        compiler_params=pltpu.CompilerParams(use_tc_tiling_on_sc=False),
        scratch_types=[
            pltpu.VMEM((NCH, LCH), jnp.int32),
            pltpu.VMEM((L, D), jnp.float32),
            pltpu.VMEM((RPW, D), jnp.float32),
            pltpu.VMEM((D,), jnp.float32),
            pltpu.SemaphoreType.DMA,
        ],
    )
    def k(tok_hbm, wt_hbm, b_hbm, out_hbm, idx_v, rows_v, out_v, bias_v, sem):
        wid = lax.axis_index("s") * NC + lax.axis_index("c")
        base = wid * RPW
        pltpu.sync_copy(b_hbm, bias_v)

        def row_body(i, carry):
            pltpu.sync_copy(tok_hbm.at[base + i], idx_v)
            for c in range(NCH):
                pltpu.async_copy(
                    wt_hbm.at[idx_v.at[c]],
                    rows_v.at[pl.ds(c * LCH, LCH)],
                    sem,
                ).wait()

            def tok_body(j, accs):
                return tuple(
                    a + rows_v[j, pl.ds(g * LANE, LANE)]
                    for g, a in enumerate(accs)
                )

            accs = tuple(bias_v[pl.ds(g * LANE, LANE)] for g in range(G))
            accs = lax.fori_loop(0, L, tok_body, accs)
            for g in range(G):
                out_v[i, pl.ds(g * LANE, LANE)] = accs[g]
            return carry

        lax.fori_loop(0, RPW, row_body, 0)
        pltpu.sync_copy(out_v, out_hbm.at[pl.ds(base, RPW)])

    return k(tok3, wt, bias)


def kernel(token_ids, W, b):
    tok3 = token_ids.astype(jnp.int32).reshape(B, NCH, LCH)
    wt = W.T
    out = _bag_kernel(tok3, wt, b)
    return out[:, None, :]


# speedup vs baseline: 8.9188x; 1.6577x over previous
import functools

import jax
import jax.numpy as jnp
from jax import lax
from jax.experimental import pallas as pl
from jax.experimental.pallas import tpu as pltpu
from jax.experimental.pallas import tpu_sc as plsc

B, L, V, D = 1024, 200, 100000, 64
LANE = 16
G = D // LANE
NC, NS = 2, 16
NW = NC * NS
RPW = B // NW
LCH = 100
NCH = L // LCH
NBUF = 2


def _bag_kernel(tok2, wt, bias):
    mesh = plsc.VectorSubcoreMesh(core_axis_name="c", subcore_axis_name="s")

    @functools.partial(
        pl.kernel,
        out_type=jax.ShapeDtypeStruct((B, D), jnp.float32),
        mesh=mesh,
        compiler_params=pltpu.CompilerParams(use_tc_tiling_on_sc=False),
        scratch_types=[
            pltpu.VMEM((RPW * NCH, LCH), jnp.int32),
            pltpu.VMEM((NBUF, L, D), jnp.float32),
            pltpu.VMEM((RPW, D), jnp.float32),
            pltpu.VMEM((D,), jnp.float32),
            pltpu.SemaphoreType.DMA,
            pltpu.SemaphoreType.DMA,
        ],
    )
    def k(tok_hbm, wt_hbm, b_hbm, out_hbm, idx_v, rows_v, out_v, bias_v,
          sem0, sem1):
        sems = (sem0, sem1)
        wid = lax.axis_index("s") * NC + lax.axis_index("c")
        base = wid * RPW
        pltpu.sync_copy(b_hbm, bias_v)
        pltpu.sync_copy(tok_hbm.at[pl.ds(base * NCH, RPW * NCH)], idx_v)

        def issue(i, s):
            for c in range(NCH):
                pltpu.async_copy(
                    wt_hbm.at[idx_v.at[i * NCH + c]],
                    rows_v.at[s, pl.ds(c * LCH, LCH)],
                    sems[s],
                )

        def drain(s):
            for c in range(NCH):
                pltpu.make_async_copy(
                    wt_hbm.at[idx_v.at[c]],
                    rows_v.at[s, pl.ds(c * LCH, LCH)],
                    sems[s],
                ).wait()

        for s in range(NBUF):
            issue(s, s)

        def pair_body(p, carry):
            for s in range(NBUF):
                i = p * NBUF + s
                drain(s)

                @pl.when(i + NBUF < RPW)
                def _():
                    issue(i + NBUF, s)

                def tok_body(j, accs):
                    return tuple(
                        a + rows_v[s, j, pl.ds(g * LANE, LANE)]
                        for g, a in enumerate(accs)
                    )

                accs = tuple(bias_v[pl.ds(g * LANE, LANE)] for g in range(G))
                accs = lax.fori_loop(0, L, tok_body, accs, unroll=8)
                for g in range(G):
                    out_v[i, pl.ds(g * LANE, LANE)] = accs[g]
            return carry

        lax.fori_loop(0, RPW // NBUF, pair_body, 0)
        pltpu.sync_copy(out_v, out_hbm.at[pl.ds(base, RPW)])

    return k(tok2, wt, bias)


def kernel(token_ids, W, b):
    tok2 = token_ids.astype(jnp.int32).reshape(B * NCH, LCH)
    wt = W.T
    out = _bag_kernel(tok2, wt, b)
    return out[:, None, :]
